# Initial kernel scaffold; baseline (speedup 1.0000x reference)
import jax
import jax.numpy as jnp
from jax.experimental import pallas as pl

_NUM_TABLE = 8
_TABLE_SIZE = 64
_CODE_LEN = 6
_P = 4


def _codes_rows(sT):
    T = _NUM_TABLE
    N = sT.shape[-1]
    planes = [sT[8 * c:8 * (c + 1), :] for c in range(_CODE_LEN)]
    absp = [jnp.abs(p) for p in planes]
    base = jnp.zeros((T, N), jnp.int32)
    for c in range(_CODE_LEN):
        base = base + (planes[c] > 0).astype(jnp.int32) * (1 << c)
    ranks = []
    for c in range(_CODE_LEN):
        r = jnp.zeros((T, N), jnp.int32)
        for c2 in range(_CODE_LEN):
            if c2 == c:
                continue
            lt = absp[c2] < absp[c]
            if c2 < c:
                lt = lt | (absp[c2] == absp[c])
            r = r + lt.astype(jnp.int32)
        ranks.append(r)
    flips = []
    for i in range(_P - 1):
        f = jnp.zeros((T, N), jnp.int32)
        for c in range(_CODE_LEN):
            f = f + (ranks[c] == i).astype(jnp.int32) * (1 << c)
        flips.append(f)
    toff = jax.lax.broadcasted_iota(jnp.int32, (T, N), 0) * _TABLE_SIZE
    rows = [base + toff]
    for i in range(_P - 1):
        rows.append(jnp.bitwise_xor(base, flips[i]) + toff)
    return rows


def _onehotT(rows, n):
    blocks = []
    for t in range(_NUM_TABLE):
        tgt = jax.lax.broadcasted_iota(jnp.int32, (_TABLE_SIZE, 1), 0) + t * _TABLE_SIZE
        b = jnp.zeros((_TABLE_SIZE, n), jnp.float32)
        for p in range(_P):
            rp = rows[p][t:t + 1, :]
            b = b + (rp == tgt).astype(jnp.float32)
        blocks.append(b)
    return jnp.concatenate(blocks, axis=0)


def _body(q_ref, k_ref, v_ref, w_ref, o_ref):
    q = q_ref[0, 0]
    k = k_ref[0, 0]
    v = v_ref[0, 0]
    w = w_ref[0]
    n = q.shape[0]
    dn_nt = (((1,), (1,)), ((), ()))
    sq = jax.lax.dot_general(w, q, dn_nt, precision=jax.lax.Precision.HIGHEST)
    sk = jax.lax.dot_general(w, k, dn_nt, precision=jax.lax.Precision.HIGHEST)
    qrows = _codes_rows(sq)
    krows = _codes_rows(sk)
    AT = _onehotT(qrows, n)
    GT = _onehotT(krows, n)
    tab = jax.lax.dot_general(AT, v, (((1,), (0,)), ((), ())))
    counts = jnp.sum(AT, axis=1, keepdims=True)
    outT = jax.lax.dot_general(tab, GT, (((0,), (0,)), ((), ())))
    cnt = jnp.sum(GT * counts, axis=0, keepdims=True)
    o_ref[0, 0] = (outT / jnp.maximum(cnt, 1.0)).T


def kernel(query_layer, key_layer, value_layer, attention_mask, projections):
    B, H, S, D = query_layer.shape
    v = value_layer * attention_mask[:, None, :, None]
    w6 = projections.transpose(0, 2, 1, 3).reshape(H, _CODE_LEN * _NUM_TABLE, D)
    out = pl.pallas_call(
        _body,
        grid=(B, H),
        in_specs=[
            pl.BlockSpec((1, 1, S, D), lambda b, h: (b, h, 0, 0)),
            pl.BlockSpec((1, 1, S, D), lambda b, h: (b, h, 0, 0)),
            pl.BlockSpec((1, 1, S, D), lambda b, h: (b, h, 0, 0)),
            pl.BlockSpec((1, _CODE_LEN * _NUM_TABLE, D), lambda b, h: (h, 0, 0)),
        ],
        out_specs=pl.BlockSpec((1, 1, S, D), lambda b, h: (b, h, 0, 0)),
        out_shape=jax.ShapeDtypeStruct((B, H, S, D), jnp.float32),
    )(query_layer, key_layer, v, w6)
    return out


# speedup vs baseline: 1182.8221x; 1182.8221x over previous
import jax
import jax.numpy as jnp
from jax.experimental import pallas as pl

_NUM_TABLE = 8
_TABLE_SIZE = 64
_CODE_LEN = 6
_P = 4


def _codes_rows(sT):
    T = _NUM_TABLE
    N = sT.shape[-1]
    planes = [sT[8 * c:8 * (c + 1), :] for c in range(_CODE_LEN)]
    absp = [jnp.abs(p) for p in planes]
    base = jnp.zeros((T, N), jnp.int32)
    for c in range(_CODE_LEN):
        base = base + (planes[c] > 0).astype(jnp.int32) * (1 << c)
    ranks = []
    for c in range(_CODE_LEN):
        r = jnp.zeros((T, N), jnp.int32)
        for c2 in range(_CODE_LEN):
            if c2 == c:
                continue
            lt = absp[c2] < absp[c]
            if c2 < c:
                lt = lt | (absp[c2] == absp[c])
            r = r + lt.astype(jnp.int32)
        ranks.append(r)
    flips = []
    for i in range(_P - 1):
        f = jnp.zeros((T, N), jnp.int32)
        for c in range(_CODE_LEN):
            f = f + (ranks[c] == i).astype(jnp.int32) * (1 << c)
        flips.append(f)
    toff = jax.lax.broadcasted_iota(jnp.int32, (T, N), 0) * _TABLE_SIZE
    rows = [base + toff]
    for i in range(_P - 1):
        rows.append(jnp.bitwise_xor(base, flips[i]) + toff)
    return rows


def _onehotT(rows, n):
    blocks = []
    for t in range(_NUM_TABLE):
        tgt = jax.lax.broadcasted_iota(jnp.int32, (_TABLE_SIZE, 1), 0) + t * _TABLE_SIZE
        b = jnp.zeros((_TABLE_SIZE, n), jnp.float32)
        for p in range(_P):
            rp = rows[p][t:t + 1, :]
            b = b + (rp == tgt).astype(jnp.float32)
        blocks.append(b)
    return jnp.concatenate(blocks, axis=0)


def _body(q_ref, k_ref, v_ref, w_ref, o_ref):
    q = q_ref[0, 0]
    k = k_ref[0, 0]
    v = v_ref[0, 0]
    w = w_ref[0]
    n = q.shape[0]
    dn_nt = (((1,), (1,)), ((), ()))
    sq = jax.lax.dot_general(w, q, dn_nt)
    sk = jax.lax.dot_general(w, k, dn_nt)
    qrows = _codes_rows(sq)
    krows = _codes_rows(sk)
    AT = _onehotT(qrows, n)
    GT = _onehotT(krows, n)
    tab = jax.lax.dot_general(AT, v, (((1,), (0,)), ((), ())))
    counts = jnp.sum(AT, axis=1, keepdims=True)
    outT = jax.lax.dot_general(tab, GT, (((0,), (0,)), ((), ())))
    cnt = jnp.sum(GT * counts, axis=0, keepdims=True)
    o_ref[0, 0] = (outT / jnp.maximum(cnt, 1.0)).T


def kernel(query_layer, key_layer, value_layer, attention_mask, projections):
    B, H, S, D = query_layer.shape
    v = value_layer * attention_mask[:, None, :, None]
    w6 = projections.transpose(0, 2, 1, 3).reshape(H, _CODE_LEN * _NUM_TABLE, D)
    out = pl.pallas_call(
        _body,
        grid=(B, H),
        in_specs=[
            pl.BlockSpec((1, 1, S, D), lambda b, h: (b, h, 0, 0)),
            pl.BlockSpec((1, 1, S, D), lambda b, h: (b, h, 0, 0)),
            pl.BlockSpec((1, 1, S, D), lambda b, h: (b, h, 0, 0)),
            pl.BlockSpec((1, _CODE_LEN * _NUM_TABLE, D), lambda b, h: (h, 0, 0)),
        ],
        out_specs=pl.BlockSpec((1, 1, S, D), lambda b, h: (b, h, 0, 0)),
        out_shape=jax.ShapeDtypeStruct((B, H, S, D), jnp.float32),
    )(query_layer, key_layer, v, w6)
    return out
